# baseline (speedup 1.0000x reference)
import jax
import jax.numpy as jnp
import numpy as np
from jax import lax
from jax.experimental import pallas as pl
from jax.experimental.pallas import tpu as pltpu

B, T, E = 1, 2048, 768
H, D = 12, 64
NE, ES, A = 8, 256, 2

BT = 256
NT = T // BT
BF = jnp.bfloat16
F32 = jnp.float32


def _rope_apply(y, cos, ssin):
    d = lax.broadcasted_iota(jnp.int32, y.shape, 1) % D
    first = d < (D // 2)
    left = jnp.concatenate([y[:, D // 2:], y[:, : D // 2]], axis=1)
    right = jnp.concatenate([y[:, -(D // 2):], y[:, : -(D // 2)]], axis=1)
    partner = jnp.where(first, left, right)
    return y * cos + partner * ssin


def _heads(y):
    return jnp.stack([y[:, h * D:(h + 1) * D] for h in range(H)], axis=0)


def _mega_kernel(x_ref, wm_ref, wr_ref, cos_ref, ssin_ref, kf_ref, vf_ref,
                 wo_ref, o_ref, ksc, vsc):
    i = pl.program_id(0)
    x = x_ref[...]
    xb = x.astype(BF)
    h = lax.dot_general(xb, wm_ref[...], (((1,), (1,)), ((), ())),
                        preferred_element_type=F32)
    qf = h[:, :E].astype(BF)
    cos = cos_ref[...]
    ssin = ssin_ref[...]
    qa = _rope_apply(h[:, E:2 * E], cos, ssin)
    ka = _rope_apply(h[:, 2 * E:3 * E], cos, ssin)
    va = h[:, 3 * E:4 * E]
    ksc[:, pl.ds(i * BT, BT), :] = _heads(ka.astype(BF))
    vsc[:, pl.ds(i * BT, BT), :] = _heads(va.astype(BF))

    logits = lax.dot_general(x, wr_ref[...], (((1,), (1,)), ((), ())),
                             preferred_element_type=F32)
    col = lax.broadcasted_iota(jnp.int32, (BT, NE), 1)
    rank = jnp.zeros((BT, NE), dtype=F32)
    for j in range(NE):
        lj = logits[:, j:j + 1]
        rank += (lj > logits).astype(F32)
        rank += ((lj == logits) & (col > j)).astype(F32)
    gates = jnp.where(rank < A, jax.nn.sigmoid(logits), 0.0)
    expand = (lax.broadcasted_iota(jnp.int32, (NE, NE * ES), 0) ==
              lax.broadcasted_iota(jnp.int32, (NE, NE * ES), 1) // ES)
    ge = jnp.dot(gates, expand.astype(F32), preferred_element_type=F32)

    qa_bf = (qa * 0.125).astype(BF)
    row_g = i * BT + lax.broadcasted_iota(jnp.int32, (BT, BT), 0)
    col_l = lax.broadcasted_iota(jnp.int32, (BT, BT), 1)
    attn_cols = []
    ffwd_cols = []
    for hh in range(H):
        qh = qa_bf[:, hh * D:(hh + 1) * D]

        def body(j, carry):
            acc, l = carry
            kh = ksc[hh, pl.ds(j * BT, BT), :]
            vh = vsc[hh, pl.ds(j * BT, BT), :]
            s = lax.dot_general(qh, kh, (((1,), (1,)), ((), ())),
                                preferred_element_type=F32)
            s = jnp.clip(s, -60.0, 60.0)
            p = jnp.where(j * BT + col_l <= row_g, jnp.exp(s), 0.0)
            l = l + jnp.sum(p, axis=1, keepdims=True)
            acc = acc + jnp.dot(p.astype(BF), vh,
                                preferred_element_type=F32)
            return acc, l

        acc0 = jnp.zeros((BT, D), dtype=F32)
        l0 = jnp.zeros((BT, 1), dtype=F32)
        acc, l = lax.fori_loop(0, i + 1, body, (acc0, l0))
        attn_cols.append((acc / l).astype(BF))

        s2 = lax.dot_general(qf[:, hh * D:(hh + 1) * D], kf_ref[hh],
                             (((1,), (1,)), ((), ())),
                             preferred_element_type=F32)
        a = 0.5 * s2 * (1.0 + lax.erf(s2 * np.float32(1.0 / np.sqrt(2.0))))
        ffwd_cols.append(jnp.dot((a * ge).astype(BF), vf_ref[hh],
                                 preferred_element_type=F32).astype(BF))

    attn = jnp.concatenate(attn_cols, axis=1)
    ffwd = jnp.concatenate(ffwd_cols, axis=1)
    wo = wo_ref[...]
    out = lax.dot_general(attn, wo[:, :E], (((1,), (1,)), ((), ())),
                          preferred_element_type=F32)
    out += lax.dot_general(ffwd, wo[:, E:], (((1,), (1,)), ((), ())),
                           preferred_element_type=F32)
    o_ref[...] = out


@jax.jit
def kernel(x, W_in, W_out, k_ffwd, v_ffwd):
    x2 = x.reshape(T, E)
    pos = np.arange(T, dtype=np.float32)
    dh = np.arange(E) % D
    inv_freq = (1.0 / (10000.0 ** (np.arange(0, D, 2, dtype=np.float32) / D)))
    ang = pos[:, None] * inv_freq[dh % (D // 2)][None, :]
    cos_t = jnp.asarray(np.cos(ang), dtype=F32)
    ssin_t = jnp.asarray(np.sin(ang) * np.where(dh < D // 2, -1.0, 1.0),
                         dtype=F32)

    w_main = W_in[:4 * E].astype(BF)
    w_r = W_in[4 * E:]
    kf2 = k_ffwd.reshape(H, NE * ES, D).astype(BF)
    vf2 = v_ffwd.reshape(H, NE * ES, D).astype(BF)
    wout_bf = W_out.astype(BF)

    out = pl.pallas_call(
        _mega_kernel,
        grid=(NT,),
        in_specs=[
            pl.BlockSpec((BT, E), lambda i: (i, 0)),
            pl.BlockSpec((4 * E, E), lambda i: (0, 0)),
            pl.BlockSpec((NE, E), lambda i: (0, 0)),
            pl.BlockSpec((BT, E), lambda i: (i, 0)),
            pl.BlockSpec((BT, E), lambda i: (i, 0)),
            pl.BlockSpec((H, NE * ES, D), lambda i: (0, 0, 0)),
            pl.BlockSpec((H, NE * ES, D), lambda i: (0, 0, 0)),
            pl.BlockSpec((E, 2 * E), lambda i: (0, 0)),
        ],
        out_specs=pl.BlockSpec((BT, E), lambda i: (i, 0)),
        out_shape=jax.ShapeDtypeStruct((T, E), F32),
        scratch_shapes=[
            pltpu.VMEM((H, T, D), BF),
            pltpu.VMEM((H, T, D), BF),
        ],
    )(x2, w_main, w_r, cos_t, ssin_t, kf2, vf2, wout_bf)

    return out.reshape(B, T, E)


# speedup vs baseline: 1.6160x; 1.6160x over previous
import jax
import jax.numpy as jnp
import numpy as np
from jax import lax
from jax.experimental import pallas as pl
from jax.experimental.pallas import tpu as pltpu

B, T, E = 1, 2048, 768
H, D = 12, 64
NE, ES, A = 8, 256, 2

BT = 256
NT = T // BT
BF = jnp.bfloat16
F32 = jnp.float32


def _rope_apply(y, cos, ssin):
    d = lax.broadcasted_iota(jnp.int32, y.shape, 1) % D
    first = d < (D // 2)
    left = jnp.concatenate([y[:, D // 2:], y[:, : D // 2]], axis=1)
    right = jnp.concatenate([y[:, -(D // 2):], y[:, : -(D // 2)]], axis=1)
    partner = jnp.where(first, left, right)
    return y * cos + partner * ssin


def _heads(y):
    return jnp.stack([y[:, h * D:(h + 1) * D] for h in range(H)], axis=0)


def _proj_kernel(x_ref, w_ref, wr_ref, cos_ref, ssin_ref,
                 qf_ref, qa_ref, ka_ref, va_ref, gates_ref):
    x = x_ref[...]
    xb = x.astype(BF)
    h = lax.dot_general(xb, w_ref[...], (((1,), (1,)), ((), ())),
                        preferred_element_type=F32)
    qf_ref[...] = h[:, :E].astype(BF)
    cos = cos_ref[...]
    ssin = ssin_ref[...]
    qa_ref[...] = _heads((_rope_apply(h[:, E:2 * E], cos, ssin)
                          * 0.125).astype(BF))
    ka_ref[...] = _heads(_rope_apply(h[:, 2 * E:3 * E], cos, ssin).astype(BF))
    va_ref[...] = _heads(h[:, 3 * E:4 * E].astype(BF))
    logits = lax.dot_general(x, wr_ref[...], (((1,), (1,)), ((), ())),
                             preferred_element_type=F32)
    col = lax.broadcasted_iota(jnp.int32, (BT, NE), 1)
    rank = jnp.zeros((BT, NE), dtype=F32)
    for j in range(NE):
        lj = logits[:, j:j + 1]
        rank += (lj > logits).astype(F32)
        rank += ((lj == logits) & (col > j)).astype(F32)
    gates_ref[...] = jnp.where(rank < A, jax.nn.sigmoid(logits), 0.0)


BA = 512


def _make_attn_kernel(qoff):
    def _attn_kernel(q_ref, k_ref, v_ref, o_ref):
        qi = pl.program_id(1) + qoff
        q = q_ref[0]
        k = k_ref[0]
        s = lax.dot_general(q, k, (((1,), (1,)), ((), ())),
                            preferred_element_type=F32)
        s = jnp.clip(s, -60.0, 60.0)
        row = qi * BA + lax.broadcasted_iota(jnp.int32, s.shape, 0)
        ccol = lax.broadcasted_iota(jnp.int32, s.shape, 1)
        p = jnp.where(ccol <= row, jnp.exp(s), 0.0)
        l = jnp.sum(p, axis=1, keepdims=True)
        acc = jnp.dot(p.astype(BF), v_ref[0], preferred_element_type=F32)
        o_ref[0] = (acc / l).astype(BF)

    return _attn_kernel


def _moe_out_kernel(qf_ref, gates_ref, alo_ref, ahi_ref, kf_ref, vf_ref,
                    w_ref, o_ref):
    is_lo = pl.program_id(0) < NT // 2
    gates = gates_ref[...]
    expand = (lax.broadcasted_iota(jnp.int32, (NE, NE * ES), 0) ==
              lax.broadcasted_iota(jnp.int32, (NE, NE * ES), 1) // ES)
    ge = jnp.dot(gates, expand.astype(F32), preferred_element_type=F32)
    ffwd_cols = []
    for h in range(H):
        qh = qf_ref[:, h * D:(h + 1) * D]
        s = lax.dot_general(qh, kf_ref[h], (((1,), (1,)), ((), ())),
                            preferred_element_type=F32)
        a = 0.5 * s * (1.0 + lax.erf(s * np.float32(1.0 / np.sqrt(2.0))))
        ffwd_cols.append(jnp.dot((a * ge).astype(BF), vf_ref[h],
                                 preferred_element_type=F32))
    ffwd = jnp.concatenate(ffwd_cols, axis=1).astype(BF)
    attn = jnp.concatenate(
        [jnp.where(is_lo, alo_ref[h], ahi_ref[h]) for h in range(H)], axis=1)
    w = w_ref[...]
    out = lax.dot_general(attn, w[:, :E], (((1,), (1,)), ((), ())),
                          preferred_element_type=F32)
    out += lax.dot_general(ffwd, w[:, E:], (((1,), (1,)), ((), ())),
                           preferred_element_type=F32)
    o_ref[...] = out


@jax.jit
def kernel(x, W_in, W_out, k_ffwd, v_ffwd):
    x2 = x.reshape(T, E)
    pos = np.arange(T, dtype=np.float32)
    dh = np.arange(E) % D
    inv_freq = (1.0 / (10000.0 ** (np.arange(0, D, 2, dtype=np.float32) / D)))
    ang = pos[:, None] * inv_freq[dh % (D // 2)][None, :]
    cos_t = jnp.asarray(np.cos(ang), dtype=F32)
    ssin_t = jnp.asarray(np.sin(ang) * np.where(dh < D // 2, -1.0, 1.0),
                         dtype=F32)

    w_main = W_in[:4 * E].astype(BF)
    w_r = W_in[4 * E:]

    qf, qa3, ka3, va3, gates = pl.pallas_call(
        _proj_kernel,
        grid=(NT,),
        in_specs=[
            pl.BlockSpec((BT, E), lambda i: (i, 0)),
            pl.BlockSpec((4 * E, E), lambda i: (0, 0)),
            pl.BlockSpec((NE, E), lambda i: (0, 0)),
            pl.BlockSpec((BT, E), lambda i: (i, 0)),
            pl.BlockSpec((BT, E), lambda i: (i, 0)),
        ],
        out_specs=[
            pl.BlockSpec((BT, E), lambda i: (i, 0)),
            pl.BlockSpec((H, BT, D), lambda i: (0, i, 0)),
            pl.BlockSpec((H, BT, D), lambda i: (0, i, 0)),
            pl.BlockSpec((H, BT, D), lambda i: (0, i, 0)),
            pl.BlockSpec((BT, NE), lambda i: (i, 0)),
        ],
        out_shape=[
            jax.ShapeDtypeStruct((T, E), BF),
            jax.ShapeDtypeStruct((H, T, D), BF),
            jax.ShapeDtypeStruct((H, T, D), BF),
            jax.ShapeDtypeStruct((H, T, D), BF),
            jax.ShapeDtypeStruct((T, NE), F32),
        ],
    )(x2, w_main, w_r, cos_t, ssin_t)

    attn_lo = pl.pallas_call(
        _make_attn_kernel(0),
        grid=(H, T // 2 // BA),
        in_specs=[
            pl.BlockSpec((1, BA, D), lambda h, qi: (h, qi, 0)),
            pl.BlockSpec((1, T // 2, D), lambda h, qi: (h, 0, 0)),
            pl.BlockSpec((1, T // 2, D), lambda h, qi: (h, 0, 0)),
        ],
        out_specs=pl.BlockSpec((1, BA, D), lambda h, qi: (h, qi, 0)),
        out_shape=jax.ShapeDtypeStruct((H, T // 2, D), BF),
    )(qa3, ka3, va3)
    attn_hi = pl.pallas_call(
        _make_attn_kernel(T // 2 // BA),
        grid=(H, T // 2 // BA),
        in_specs=[
            pl.BlockSpec((1, BA, D),
                         lambda h, qi: (h, qi + T // 2 // BA, 0)),
            pl.BlockSpec((1, T, D), lambda h, qi: (h, 0, 0)),
            pl.BlockSpec((1, T, D), lambda h, qi: (h, 0, 0)),
        ],
        out_specs=pl.BlockSpec((1, BA, D), lambda h, qi: (h, qi, 0)),
        out_shape=jax.ShapeDtypeStruct((H, T // 2, D), BF),
    )(qa3, ka3, va3)

    kf2 = k_ffwd.reshape(H, NE * ES, D).astype(BF)
    vf2 = v_ffwd.reshape(H, NE * ES, D).astype(BF)
    wout_bf = W_out.astype(BF)
    out = pl.pallas_call(
        _moe_out_kernel,
        grid=(NT,),
        in_specs=[
            pl.BlockSpec((BT, E), lambda i: (i, 0)),
            pl.BlockSpec((BT, NE), lambda i: (i, 0)),
            pl.BlockSpec((H, BT, D),
                         lambda i: (0, jnp.minimum(i, NT // 2 - 1), 0)),
            pl.BlockSpec((H, BT, D),
                         lambda i: (0, jnp.maximum(i - NT // 2, 0), 0)),
            pl.BlockSpec((H, NE * ES, D), lambda i: (0, 0, 0)),
            pl.BlockSpec((H, NE * ES, D), lambda i: (0, 0, 0)),
            pl.BlockSpec((E, 2 * E), lambda i: (0, 0)),
        ],
        out_specs=pl.BlockSpec((BT, E), lambda i: (i, 0)),
        out_shape=jax.ShapeDtypeStruct((T, E), F32),
    )(qf, gates, attn_lo, attn_hi, kf2, vf2, wout_bf)

    return out.reshape(B, T, E)
